# Initial kernel scaffold; baseline (speedup 1.0000x reference)
import jax
import jax.numpy as jnp
from jax.experimental import pallas as pl


def kernel(x1, x2, mask):
    raise NotImplementedError("write your pallas kernel here")


# speedup vs baseline: 1.1025x; 1.1025x over previous
import jax
import jax.numpy as jnp
import numpy as np
from jax.experimental import pallas as pl
from jax.experimental.pallas import tpu as pltpu

_BINS = 256
_MIN_V = -0.25
_MAX_V = 1.25
_XSIGMA = 1.0 / float(np.sqrt(2.0 * np.pi))
_DELTA = (_MAX_V - _MIN_V) / _BINS
_SIGMA = _DELTA * _XSIGMA
_COEF = float(_DELTA / (np.sqrt(2.0 * np.pi) * _SIGMA))
_LOG2E = float(np.log2(np.e))
_A = float(np.sqrt(0.5 * _LOG2E) / _SIGMA)

_CK = 2048


def _ghist_kernel(x1_ref, x2_ref, m_ref, o_ref):
    k = pl.program_id(1)
    u1 = x1_ref[0] * _A
    u2 = x2_ref[0] * _A
    lm = jnp.log2(m_ref[0])
    i = jax.lax.broadcasted_iota(jnp.int32, (_BINS, 1), 0).astype(jnp.float32)
    v = (_MIN_V + _DELTA * (i + 0.5)) * _A
    w1 = u1 - v
    w2 = u2 - v
    g1 = jnp.exp2(-(w1 * w1))
    g2 = jnp.exp2(lm - w2 * w2)
    res = jax.lax.dot_general(
        g1, g2, (((1,), (1,)), ((), ())),
        preferred_element_type=jnp.float32)
    if _COEF != 1.0:
        res = res * _COEF

    @pl.when(k == 0)
    def _():
        o_ref[...] = res[None]

    @pl.when(k != 0)
    def _():
        o_ref[...] += res[None]


def kernel(x1, x2, mask, *, interpret=False):
    B, N = x1.shape
    nk = N // _CK
    x1r = x1.reshape(B * nk, 1, _CK)
    x2r = x2.reshape(B * nk, 1, _CK)
    mr = mask.reshape(B * nk, 1, _CK)
    spec = pl.BlockSpec((1, 1, _CK), lambda b, k: (b * nk + k, 0, 0))
    return pl.pallas_call(
        _ghist_kernel,
        grid=(B, nk),
        in_specs=[spec, spec, spec],
        out_specs=pl.BlockSpec((1, _BINS, _BINS), lambda b, k: (b, 0, 0)),
        out_shape=jax.ShapeDtypeStruct((B, _BINS, _BINS), jnp.float32),
        compiler_params=pltpu.CompilerParams(
            dimension_semantics=("parallel", "arbitrary"),
        ),
        name="gaussian_hist2d",
        interpret=interpret,
    )(x1r, x2r, mr)


# speedup vs baseline: 1.4315x; 1.2984x over previous
import jax
import jax.numpy as jnp
import numpy as np
from jax.experimental import pallas as pl
from jax.experimental.pallas import tpu as pltpu

_BINS = 256
_MIN_V = -0.25
_MAX_V = 1.25
_XSIGMA = 1.0 / float(np.sqrt(2.0 * np.pi))
_DELTA = (_MAX_V - _MIN_V) / _BINS
_SIGMA = _DELTA * _XSIGMA
_COEF = float(_DELTA / (np.sqrt(2.0 * np.pi) * _SIGMA))
_LOG2E = float(np.log2(np.e))
_A = float(np.sqrt(0.5 * _LOG2E) / _SIGMA)

_CK = 16384


def _ghist_kernel(x1_ref, x2_ref, m_ref, o_ref):
    k = pl.program_id(1)
    u1 = x1_ref[0] * _A
    u2 = x2_ref[0] * _A
    lm = jnp.log2(m_ref[0])
    i = jax.lax.broadcasted_iota(jnp.int32, (_BINS, 1), 0).astype(jnp.float32)
    v = (_MIN_V + _DELTA * (i + 0.5)) * _A
    w1 = u1 - v
    w2 = u2 - v
    g1 = jnp.exp2(-(w1 * w1))
    g2 = jnp.exp2(lm - w2 * w2)
    res = jax.lax.dot_general(
        g1, g2, (((1,), (1,)), ((), ())),
        preferred_element_type=jnp.float32)
    if _COEF != 1.0:
        res = res * _COEF

    @pl.when(k == 0)
    def _():
        o_ref[...] = res[None]

    @pl.when(k != 0)
    def _():
        o_ref[...] += res[None]


def kernel(x1, x2, mask, *, interpret=False):
    B, N = x1.shape
    nk = N // _CK
    x1r = x1.reshape(B * nk, 1, _CK)
    x2r = x2.reshape(B * nk, 1, _CK)
    mr = mask.reshape(B * nk, 1, _CK)
    spec = pl.BlockSpec((1, 1, _CK), lambda b, k: (b * nk + k, 0, 0))
    return pl.pallas_call(
        _ghist_kernel,
        grid=(B, nk),
        in_specs=[spec, spec, spec],
        out_specs=pl.BlockSpec((1, _BINS, _BINS), lambda b, k: (b, 0, 0)),
        out_shape=jax.ShapeDtypeStruct((B, _BINS, _BINS), jnp.float32),
        compiler_params=pltpu.CompilerParams(
            dimension_semantics=("parallel", "arbitrary"),
        ),
        name="gaussian_hist2d",
        interpret=interpret,
    )(x1r, x2r, mr)


# speedup vs baseline: 1.4542x; 1.0159x over previous
import jax
import jax.numpy as jnp
import numpy as np
from jax.experimental import pallas as pl
from jax.experimental.pallas import tpu as pltpu

_BINS = 256
_MIN_V = -0.25
_MAX_V = 1.25
_XSIGMA = 1.0 / float(np.sqrt(2.0 * np.pi))
_DELTA = (_MAX_V - _MIN_V) / _BINS
_SIGMA = _DELTA * _XSIGMA
_COEF = float(_DELTA / (np.sqrt(2.0 * np.pi) * _SIGMA))
_LOG2E = float(np.log2(np.e))
_A = float(np.sqrt(0.5 * _LOG2E) / _SIGMA)

_CK = 8192


def _ghist_kernel(x1_ref, x2_ref, m_ref, o_ref):
    k = pl.program_id(1)
    u1 = x1_ref[0] * _A
    u2 = x2_ref[0] * _A
    lm = jnp.log2(m_ref[0]).astype(jnp.bfloat16)
    i = jax.lax.broadcasted_iota(jnp.int32, (_BINS, 1), 0).astype(jnp.float32)
    v = (_MIN_V + _DELTA * (i + 0.5)) * _A
    w1 = (u1 - v).astype(jnp.bfloat16)
    w2 = (u2 - v).astype(jnp.bfloat16)
    g1 = jnp.exp2(-(w1 * w1))
    g2 = jnp.exp2(lm - w2 * w2)
    res = jax.lax.dot_general(
        g1, g2, (((1,), (1,)), ((), ())),
        preferred_element_type=jnp.float32)
    if _COEF != 1.0:
        res = res * _COEF

    @pl.when(k == 0)
    def _():
        o_ref[...] = res[None]

    @pl.when(k != 0)
    def _():
        o_ref[...] += res[None]


def kernel(x1, x2, mask, *, interpret=False):
    B, N = x1.shape
    nk = N // _CK
    x1r = x1.reshape(B * nk, 1, _CK)
    x2r = x2.reshape(B * nk, 1, _CK)
    mr = mask.reshape(B * nk, 1, _CK)
    spec = pl.BlockSpec((1, 1, _CK), lambda b, k: (b * nk + k, 0, 0))
    return pl.pallas_call(
        _ghist_kernel,
        grid=(B, nk),
        in_specs=[spec, spec, spec],
        out_specs=pl.BlockSpec((1, _BINS, _BINS), lambda b, k: (b, 0, 0)),
        out_shape=jax.ShapeDtypeStruct((B, _BINS, _BINS), jnp.float32),
        compiler_params=pltpu.CompilerParams(
            dimension_semantics=("parallel", "arbitrary"),
        ),
        name="gaussian_hist2d",
        interpret=interpret,
    )(x1r, x2r, mr)


# speedup vs baseline: 1.5161x; 1.0425x over previous
import jax
import jax.numpy as jnp
import numpy as np
from jax.experimental import pallas as pl
from jax.experimental.pallas import tpu as pltpu

_BINS = 256
_MIN_V = -0.25
_MAX_V = 1.25
_XSIGMA = 1.0 / float(np.sqrt(2.0 * np.pi))
_DELTA = (_MAX_V - _MIN_V) / _BINS
_SIGMA = _DELTA * _XSIGMA
_COEF = float(_DELTA / (np.sqrt(2.0 * np.pi) * _SIGMA))
_LOG2E = float(np.log2(np.e))
_A = float(np.sqrt(0.5 * _LOG2E) / _SIGMA)

_CK = 16384


def _ghist_kernel(x1_ref, x2_ref, m_ref, o_ref):
    k = pl.program_id(1)
    u1 = x1_ref[0] * _A
    u2 = x2_ref[0] * _A
    lm = jnp.log2(m_ref[0]).astype(jnp.bfloat16)
    i = jax.lax.broadcasted_iota(jnp.int32, (_BINS, 1), 0).astype(jnp.float32)
    v = (_MIN_V + _DELTA * (i + 0.5)) * _A
    w1 = (u1 - v).astype(jnp.bfloat16)
    w2 = (u2 - v).astype(jnp.bfloat16)
    g1 = jnp.exp2(-(w1 * w1))
    g2 = jnp.exp2(lm - w2 * w2)
    res = jax.lax.dot_general(
        g1, g2, (((1,), (1,)), ((), ())),
        preferred_element_type=jnp.float32)
    if _COEF != 1.0:
        res = res * _COEF

    @pl.when(k == 0)
    def _():
        o_ref[...] = res[None]

    @pl.when(k != 0)
    def _():
        o_ref[...] += res[None]


def kernel(x1, x2, mask, *, interpret=False):
    B, N = x1.shape
    nk = N // _CK
    x1r = x1.reshape(B * nk, 1, _CK)
    x2r = x2.reshape(B * nk, 1, _CK)
    mr = mask.reshape(B * nk, 1, _CK)
    spec = pl.BlockSpec((1, 1, _CK), lambda b, k: (b * nk + k, 0, 0))
    return pl.pallas_call(
        _ghist_kernel,
        grid=(B, nk),
        in_specs=[spec, spec, spec],
        out_specs=pl.BlockSpec((1, _BINS, _BINS), lambda b, k: (b, 0, 0)),
        out_shape=jax.ShapeDtypeStruct((B, _BINS, _BINS), jnp.float32),
        compiler_params=pltpu.CompilerParams(
            dimension_semantics=("parallel", "arbitrary"),
        ),
        name="gaussian_hist2d",
        interpret=interpret,
    )(x1r, x2r, mr)


# speedup vs baseline: 1.5195x; 1.0023x over previous
import jax
import jax.numpy as jnp
import numpy as np
from jax.experimental import pallas as pl
from jax.experimental.pallas import tpu as pltpu

_BINS = 256
_MIN_V = -0.25
_MAX_V = 1.25
_XSIGMA = 1.0 / float(np.sqrt(2.0 * np.pi))
_DELTA = (_MAX_V - _MIN_V) / _BINS
_SIGMA = _DELTA * _XSIGMA
_COEF = float(_DELTA / (np.sqrt(2.0 * np.pi) * _SIGMA))
_LOG2E = float(np.log2(np.e))
_A = float(np.sqrt(0.5 * _LOG2E) / _SIGMA)

_CK = 16384


def _ghist_kernel(x1_ref, x2_ref, m_ref, o_ref):
    k = pl.program_id(1)
    u1 = x1_ref[0] * _A
    u2 = x2_ref[0] * _A
    lm = jnp.log2(m_ref[0]).astype(jnp.bfloat16)
    i = jax.lax.broadcasted_iota(jnp.int32, (_BINS, 1), 0).astype(jnp.float32)
    v = (_MIN_V + _DELTA * (i + 0.5)) * _A
    w1 = (u1 - v).astype(jnp.bfloat16)
    w2 = (u2 - v).astype(jnp.bfloat16)
    g1 = jnp.exp2(-(w1 * w1))
    g2 = jnp.exp2(lm - w2 * w2)
    res = jax.lax.dot_general(
        g1, g2, (((1,), (1,)), ((), ())),
        preferred_element_type=jnp.float32)
    if _COEF != 1.0:
        res = res * _COEF

    o_ref[...] = jnp.where(k == 0, res[None], res[None] + o_ref[...])


def kernel(x1, x2, mask, *, interpret=False):
    B, N = x1.shape
    nk = N // _CK
    x1r = x1.reshape(B * nk, 1, _CK)
    x2r = x2.reshape(B * nk, 1, _CK)
    mr = mask.reshape(B * nk, 1, _CK)
    spec = pl.BlockSpec((1, 1, _CK), lambda b, k: (b * nk + k, 0, 0))
    return pl.pallas_call(
        _ghist_kernel,
        grid=(B, nk),
        in_specs=[spec, spec, spec],
        out_specs=pl.BlockSpec((1, _BINS, _BINS), lambda b, k: (b, 0, 0)),
        out_shape=jax.ShapeDtypeStruct((B, _BINS, _BINS), jnp.float32),
        compiler_params=pltpu.CompilerParams(
            dimension_semantics=("parallel", "arbitrary"),
        ),
        name="gaussian_hist2d",
        interpret=interpret,
    )(x1r, x2r, mr)


# speedup vs baseline: 1.5199x; 1.0002x over previous
import jax
import jax.numpy as jnp
import numpy as np
from jax.experimental import pallas as pl
from jax.experimental.pallas import tpu as pltpu

_BINS = 256
_MIN_V = -0.25
_MAX_V = 1.25
_XSIGMA = 1.0 / float(np.sqrt(2.0 * np.pi))
_DELTA = (_MAX_V - _MIN_V) / _BINS
_SIGMA = _DELTA * _XSIGMA
_COEF = float(_DELTA / (np.sqrt(2.0 * np.pi) * _SIGMA))
_LOG2E = float(np.log2(np.e))
_A = float(np.sqrt(0.5 * _LOG2E) / _SIGMA)

_CK = 16384


def _ghist_kernel(x1_ref, x2_ref, m_ref, o_ref):
    k = pl.program_id(1)
    u1 = x1_ref[0] * _A
    u2 = x2_ref[0] * _A
    lm = jnp.log2(m_ref[0]).astype(jnp.bfloat16)
    i = jax.lax.broadcasted_iota(jnp.int32, (_BINS, 1), 0).astype(jnp.float32)
    v = (_MIN_V + _DELTA * (i + 0.5)) * _A
    w1 = (u1 - v).astype(jnp.bfloat16)
    w2 = (u2 - v).astype(jnp.bfloat16)
    g1 = jnp.exp2(-(w1 * w1))
    g2 = jnp.exp2(lm - w2 * w2)
    res = jax.lax.dot_general(
        g1, g2, (((1,), (1,)), ((), ())),
        preferred_element_type=jnp.float32)
    if _COEF != 1.0:
        res = res * _COEF

    o_ref[...] = jnp.where(k == 0, res[None], res[None] + o_ref[...])


def kernel(x1, x2, mask):
    B, N = x1.shape
    nk = N // _CK
    x1r = x1.reshape(B * nk, 1, _CK)
    x2r = x2.reshape(B * nk, 1, _CK)
    mr = mask.reshape(B * nk, 1, _CK)
    spec = pl.BlockSpec((1, 1, _CK), lambda b, k: (b * nk + k, 0, 0))
    return pl.pallas_call(
        _ghist_kernel,
        grid=(B, nk),
        in_specs=[spec, spec, spec],
        out_specs=pl.BlockSpec((1, _BINS, _BINS), lambda b, k: (b, 0, 0)),
        out_shape=jax.ShapeDtypeStruct((B, _BINS, _BINS), jnp.float32),
        compiler_params=pltpu.CompilerParams(
            dimension_semantics=("parallel", "arbitrary"),
        ),
        name="gaussian_hist2d",
    )(x1r, x2r, mr)
